# Initial kernel scaffold; baseline (speedup 1.0000x reference)
import jax
import jax.numpy as jnp
from jax.experimental import pallas as pl


def kernel(x, topo, edge_index, W1, a_src1, a_dst1, b1, W2, a_src2, a_dst2, b2):
    raise NotImplementedError("write your pallas kernel here")


# speedup vs baseline: 32.5419x; 32.5419x over previous
import functools

import jax
import jax.numpy as jnp
from jax import lax
from jax.experimental import pallas as pl
from jax.experimental.pallas import tpu as pltpu
from jax.experimental.pallas import tpu_sc as plsc

NC = 2
NS = 16
LANES = 16


def _pick_chunk(ew: int) -> int:
    for k in (128, 120, 112, 104, 96, 88, 80, 72, 64, 56, 48, 40, 32, 24, 16, 8):
        if ew % k == 0:
            return k
    return ew


def _make_edge_pass(n, e, hc, interpret=False):
    k = _pick_chunk(e // (NC * NS))
    ew = e // (NC * NS)
    chunks = ew // k
    assert chunks * k * NC * NS == e
    assert n % NS == 0
    nrows = n // NS
    nj = hc // LANES
    assert hc % LANES == 0

    mesh = plsc.VectorSubcoreMesh(
        core_axis_name="c", subcore_axis_name="s", num_cores=NC,
        num_subcores=NS)

    @functools.partial(
        pl.kernel,
        out_type=jax.ShapeDtypeStruct((NC, n, 2 * hc), jnp.float32),
        mesh=mesh,
        scratch_types=[
            pltpu.VMEM((k, 2 * hc), jnp.float32),
            pltpu.VMEM((k, hc), jnp.float32),
            pltpu.VMEM((k, 2 * hc), jnp.float32),
            pltpu.VMEM((k,), jnp.int32),
            pltpu.VMEM((k,), jnp.int32),
            pltpu.VMEM_SHARED((n, 2 * hc), jnp.float32),
        ],
        compiler_params=pltpu.CompilerParams(use_tc_tiling_on_sc=False),
        interpret=interpret,
    )
    def edge_pass(tsrc, tdst, src, dst, zeros, out,
                  hs_v, ad_v, msg_v, src_v, dst_v, accum):
        cid = lax.axis_index("c")
        sid = lax.axis_index("s")
        wid = cid * NS + sid
        row0 = sid * nrows

        pltpu.sync_copy(zeros.at[pl.ds(row0, nrows)],
                        accum.at[pl.ds(row0, nrows)])
        plsc.subcore_barrier()

        def chunk_body(c, _):
            base = wid * ew + c * k
            pltpu.sync_copy(src.at[pl.ds(base, k)], src_v)
            pltpu.sync_copy(dst.at[pl.ds(base, k)], dst_v)
            pltpu.sync_copy(tsrc.at[src_v], hs_v)
            pltpu.sync_copy(tdst.at[dst_v], ad_v)

            def edge_body(t, _):
                for j in range(nj):
                    a = hs_v[t, pl.ds(hc + j * LANES, LANES)]
                    d = ad_v[t, pl.ds(j * LANES, LANES)]
                    x = a + d
                    ex = jnp.exp(jnp.maximum(x, 0.2 * x))
                    h = hs_v[t, pl.ds(j * LANES, LANES)]
                    msg_v[t, pl.ds(j * LANES, LANES)] = h * ex
                    msg_v[t, pl.ds(hc + j * LANES, LANES)] = ex
                return _

            lax.fori_loop(0, k, edge_body, None)
            pltpu.sync_copy(msg_v, accum.at[dst_v], add=True)
            return _

        lax.fori_loop(0, chunks, chunk_body, None)
        plsc.subcore_barrier()

        pltpu.sync_copy(accum.at[pl.ds(row0, nrows)],
                        out.at[cid, pl.ds(row0, nrows)])

    return edge_pass


def _tc_input_kernel(x_ref, topo_ref, wsrc_ref, wdst_ref, tsrc_ref, tdst_ref):
    xin = jnp.concatenate([x_ref[...], topo_ref[...]], axis=1)
    tsrc_ref[...] = jnp.dot(xin, wsrc_ref[...],
                            preferred_element_type=jnp.float32)
    tdst_ref[...] = jnp.dot(xin, wdst_ref[...],
                            preferred_element_type=jnp.float32)


def _tc_mid_kernel(p0_ref, p1_ref, b1_ref, wsrc_ref, wdst_ref,
                   tsrc_ref, tdst_ref, *, hc):
    s = p0_ref[...] + p1_ref[...]
    num = s[:, :hc]
    den = s[:, hc:]
    h1 = num / (den + 1e-16) + b1_ref[...]
    h1 = jnp.where(h1 > 0, h1, jnp.exp(jnp.minimum(h1, 0.0)) - 1.0)
    tsrc_ref[...] = jnp.dot(h1, wsrc_ref[...],
                            preferred_element_type=jnp.float32)
    tdst_ref[...] = jnp.dot(h1, wdst_ref[...],
                            preferred_element_type=jnp.float32)


def _tc_out_kernel(p0_ref, p1_ref, b2_ref, o_ref, *, hc):
    s = p0_ref[...] + p1_ref[...]
    num = s[:, :hc]
    den = s[:, hc:]
    h2 = num / (den + 1e-16) + b2_ref[...]
    m = jnp.max(h2, axis=1, keepdims=True)
    z = h2 - m
    lse = jnp.log(jnp.sum(jnp.exp(z), axis=1, keepdims=True))
    o_ref[...] = z - lse


def _expand_weights(w, a_src, a_dst, heads, out_ch):
    blk = jnp.kron(jnp.eye(heads, dtype=w.dtype),
                   jnp.ones((out_ch, out_ch), dtype=w.dtype))
    asrc_m = blk * a_src.reshape(-1)[:, None]
    adst_m = blk * a_dst.reshape(-1)[:, None]
    wsrc = jnp.concatenate([w, w @ asrc_m], axis=1)
    wdst = w @ adst_m
    return wsrc, wdst


def kernel(x, topo, edge_index, W1, a_src1, a_dst1, b1, W2, a_src2, a_dst2,
           b2, *, interpret=False):
    n = x.shape[0]
    e = edge_index.shape[1]
    h1, c1 = a_src1.shape
    hc1 = h1 * c1
    hc2 = a_src2.shape[0] * a_src2.shape[1]

    src = edge_index[0].astype(jnp.int32)
    dst = edge_index[1].astype(jnp.int32)

    w1src, w1dst = _expand_weights(W1, a_src1, a_dst1, h1, c1)
    w2src, w2dst = _expand_weights(W2, a_src2, a_dst2, 1, hc2)

    zeros1 = jnp.zeros((n, 2 * hc1), jnp.float32)
    zeros2 = jnp.zeros((n, 2 * hc2), jnp.float32)

    grid = 1
    for g in (5, 4, 2):
        if n % g == 0 and (n // g) % 8 == 0:
            grid = g
            break
    bn = n // grid

    tsrc1, tdst1 = pl.pallas_call(
        _tc_input_kernel,
        grid=(grid,),
        in_specs=[
            pl.BlockSpec((bn, x.shape[1]), lambda i: (i, 0)),
            pl.BlockSpec((bn, topo.shape[1]), lambda i: (i, 0)),
            pl.BlockSpec(w1src.shape, lambda i: (0, 0)),
            pl.BlockSpec(w1dst.shape, lambda i: (0, 0)),
        ],
        out_specs=[
            pl.BlockSpec((bn, 2 * hc1), lambda i: (i, 0)),
            pl.BlockSpec((bn, hc1), lambda i: (i, 0)),
        ],
        out_shape=[
            jax.ShapeDtypeStruct((n, 2 * hc1), jnp.float32),
            jax.ShapeDtypeStruct((n, hc1), jnp.float32),
        ],
        interpret=interpret,
    )(x, topo, w1src, w1dst)

    edge1 = _make_edge_pass(n, e, hc1, interpret=interpret)
    p1 = edge1(tsrc1, tdst1, src, dst, zeros1)

    tsrc2, tdst2 = pl.pallas_call(
        functools.partial(_tc_mid_kernel, hc=hc1),
        grid=(grid,),
        in_specs=[
            pl.BlockSpec((bn, 2 * hc1), lambda i: (i, 0)),
            pl.BlockSpec((bn, 2 * hc1), lambda i: (i, 0)),
            pl.BlockSpec(b1.shape, lambda i: (0,)),
            pl.BlockSpec(w2src.shape, lambda i: (0, 0)),
            pl.BlockSpec(w2dst.shape, lambda i: (0, 0)),
        ],
        out_specs=[
            pl.BlockSpec((bn, 2 * hc2), lambda i: (i, 0)),
            pl.BlockSpec((bn, hc2), lambda i: (i, 0)),
        ],
        out_shape=[
            jax.ShapeDtypeStruct((n, 2 * hc2), jnp.float32),
            jax.ShapeDtypeStruct((n, hc2), jnp.float32),
        ],
        interpret=interpret,
    )(p1[0], p1[1], b1, w2src, w2dst)

    edge2 = _make_edge_pass(n, e, hc2, interpret=interpret)
    p2 = edge2(tsrc2, tdst2, src, dst, zeros2)

    out = pl.pallas_call(
        functools.partial(_tc_out_kernel, hc=hc2),
        grid=(grid,),
        in_specs=[
            pl.BlockSpec((bn, 2 * hc2), lambda i: (i, 0)),
            pl.BlockSpec((bn, 2 * hc2), lambda i: (i, 0)),
            pl.BlockSpec(b2.shape, lambda i: (0,)),
        ],
        out_specs=pl.BlockSpec((bn, hc2), lambda i: (i, 0)),
        out_shape=jax.ShapeDtypeStruct((n, hc2), jnp.float32),
        interpret=interpret,
    )(p2[0], p2[1], b2)

    return out
